# baseline (speedup 1.0000x reference)
import functools

import jax
import jax.numpy as jnp
from jax import lax
from jax.experimental import pallas as pl
from jax.experimental.pallas import tpu as pltpu
from jax.experimental.pallas import tpu_sc as plsc

B = 16384
D = 64
L = 16
NC, NS = 2, 16
NW = NC * NS
BPW = B // NW
CH = 128
NCH = BPW // CH

_mesh = plsc.VectorSubcoreMesh(core_axis_name="c", subcore_axis_name="s")


@functools.partial(
    pl.kernel,
    out_type=jax.ShapeDtypeStruct((B,), jnp.float32),
    mesh=_mesh,
    compiler_params=pltpu.CompilerParams(needs_layout_passes=False,
                                         use_tc_tiling_on_sc=False),
    scratch_types=[
        pltpu.VMEM((NCH, CH), jnp.int32),
        pltpu.VMEM((NCH, CH), jnp.int32),
        pltpu.VMEM((NCH, CH), jnp.int32),
        pltpu.VMEM((BPW, D), jnp.float32),
        pltpu.VMEM((BPW, D), jnp.float32),
        pltpu.VMEM((BPW, D), jnp.float32),
        pltpu.VMEM((BPW,), jnp.float32),
        pltpu.SemaphoreType.DMA,
    ],
)
def _sc_score(head_hbm, rel_hbm, tail_hbm, ent_hbm, relemb_hbm, out_hbm,
              hidx, ridx, tidx, hrow, rrow, trow, outv, sem):
    wid = lax.axis_index("s") * NC + lax.axis_index("c")
    base = pl.multiple_of(wid * BPW, BPW)

    for j in range(NCH):
        pltpu.sync_copy(head_hbm.at[pl.ds(base + j * CH, CH)], hidx.at[j])
        pltpu.sync_copy(rel_hbm.at[pl.ds(base + j * CH, CH)], ridx.at[j])
        pltpu.sync_copy(tail_hbm.at[pl.ds(base + j * CH, CH)], tidx.at[j])

    copies = []
    for j in range(NCH):
        sl = pl.ds(j * CH, CH)
        copies.append(pltpu.async_copy(ent_hbm.at[hidx.at[j]], hrow.at[sl], sem))
        copies.append(pltpu.async_copy(relemb_hbm.at[ridx.at[j]], rrow.at[sl], sem))
        copies.append(pltpu.async_copy(ent_hbm.at[tidx.at[j]], trow.at[sl], sem))
    for c in copies:
        c.wait()

    def group_body(g, carry):
        rows = lax.iota(jnp.int32, L) + g * L

        def col_body(d, acc):
            col = jnp.full((L,), d, jnp.int32)
            hv = plsc.load_gather(hrow, [rows, col])
            rv = plsc.load_gather(rrow, [rows, col])
            tv = plsc.load_gather(trow, [rows, col])
            e = hv + rv - tv
            return acc + e * e

        x = lax.fori_loop(0, D, col_body, jnp.zeros((L,), jnp.float32)) + 1e-12
        i = lax.bitcast_convert_type(x, jnp.int32)
        r = lax.bitcast_convert_type(jnp.int32(0x5F3759DF) - (i >> 1),
                                     jnp.float32)
        for _ in range(3):
            r = r * (1.5 - 0.5 * x * r * r)
        outv[pl.ds(pl.multiple_of(g * L, L), L)] = -(x * r)
        return carry

    lax.fori_loop(0, BPW // L, group_body, 0)
    pltpu.sync_copy(outv, out_hbm.at[pl.ds(base, BPW)])


def kernel(head, relation, tail, entity_embeddings, relation_embeddings):
    return _sc_score(head.astype(jnp.int32), relation.astype(jnp.int32),
                     tail.astype(jnp.int32), entity_embeddings,
                     relation_embeddings)
